# baseline (speedup 1.0000x reference)
import functools

import jax
import jax.numpy as jnp
from jax import lax
from jax.experimental import pallas as pl
from jax.experimental.pallas import tpu as pltpu
from jax.experimental.pallas import tpu_sc as plsc

PATCH = 16
IMG = 384
NPR = IMG // PATCH
EMB = PATCH * PATCH * 3
ROWS_PAD = 240


def _sc_gather(weight, idx):
    n_chunks = idx.shape[0]
    info = plsc.get_sparse_core_info()
    nc, ns = info.num_cores, info.num_subcores
    nw = nc * ns
    chunks_per_w = n_chunks // nw
    rows_per_w = chunks_per_w * 120

    mesh = plsc.VectorSubcoreMesh(core_axis_name="c", subcore_axis_name="s")

    @functools.partial(
        pl.kernel,
        mesh=mesh,
        out_type=jax.ShapeDtypeStruct((n_chunks * 120, EMB), jnp.float32),
        scratch_types=[
            pltpu.VMEM((chunks_per_w, 120), jnp.int32),
            pltpu.VMEM((120, EMB), jnp.float32),
            pltpu.SemaphoreType.DMA,
        ],
    )
    def k(weight_hbm, idx_hbm, out_hbm, idx_v, rows_v, sem):
        wid = lax.axis_index("s") * nc + lax.axis_index("c")
        base = wid * rows_per_w
        pltpu.sync_copy(idx_hbm.at[pl.ds(wid * chunks_per_w, chunks_per_w)], idx_v)
        for c in range(chunks_per_w):
            pltpu.async_copy(weight_hbm.at[idx_v.at[c]], rows_v, sem).wait()
            pltpu.sync_copy(rows_v, out_hbm.at[pl.ds(base + c * 120, 120)])

    return k(weight, idx)


def _tc_assemble(emb):
    n_batch = emb.shape[0]

    def body(emb_ref, out_ref, acc_ref):
        n = pl.program_id(0)
        i = pl.program_id(1)
        t = jnp.tanh(emb_ref[0])
        blk = t.reshape(NPR, 3 * PATCH, PATCH).transpose(1, 0, 2)
        out_ref[0] = blk.reshape(3, PATCH, IMG)
        s = jnp.sum(t * t)
        first = jnp.logical_and(n == 0, i == 0)
        acc_ref[0, 0] = jnp.where(first, 0.0, acc_ref[0, 0]) + s

    img, ssq = pl.pallas_call(
        body,
        grid=(n_batch, NPR),
        in_specs=[
            pl.BlockSpec((1, NPR, EMB), lambda n, i: (n, jnp.minimum(i, 9), 0)),
        ],
        out_specs=(
            pl.BlockSpec((1, 3, PATCH, IMG), lambda n, i: (n, 0, i, 0)),
            pl.BlockSpec((1, 1), lambda n, i: (0, 0), memory_space=pltpu.SMEM),
        ),
        out_shape=(
            jax.ShapeDtypeStruct((n_batch, 3, IMG, IMG), jnp.float32),
            jax.ShapeDtypeStruct((1, 1), jnp.float32),
        ),
    )(emb)
    return img, ssq


def kernel(sentence_batch, weight):
    n_batch, seq_l = sentence_batch.shape
    tokens = sentence_batch.astype(jnp.int32)
    row_src = jnp.minimum(jnp.arange(ROWS_PAD), seq_l - 1)
    idx = jnp.take(tokens, row_src, axis=1)
    idx = idx.reshape(n_batch * ROWS_PAD // 120, 120)
    emb_raw = _sc_gather(weight, idx)
    emb = emb_raw.reshape(n_batch, ROWS_PAD, EMB)
    img, ssq = _tc_assemble(emb)
    pert_norm = jnp.sqrt(ssq[0, 0]) / n_batch
    return img, pert_norm


# speedup vs baseline: 3.9622x; 3.9622x over previous
import functools

import jax
import jax.numpy as jnp
from jax import lax
from jax.experimental import pallas as pl
from jax.experimental.pallas import tpu as pltpu
from jax.experimental.pallas import tpu_sc as plsc

PATCH = 16
IMG = 384
NPR = IMG // PATCH
EMB = PATCH * PATCH * 3
N_BATCH = 64
SEQ_L = 200

_INFO = plsc.get_sparse_core_info()
_NC, _NS = _INFO.num_cores, _INFO.num_subcores
_NW = _NC * _NS


_GCHUNK = 80
_NCHUNK = (N_BATCH * SEQ_L) // (_NW * _GCHUNK)


def _sc_gather(weight, idx):
    n_rows = idx.shape[0] * idx.shape[1] * idx.shape[2]
    rows_per_w = _NCHUNK * _GCHUNK

    mesh = plsc.VectorSubcoreMesh(core_axis_name="c", subcore_axis_name="s")

    @functools.partial(
        pl.kernel,
        mesh=mesh,
        out_type=jax.ShapeDtypeStruct((n_rows, EMB), jnp.float32),
        scratch_types=[
            pltpu.VMEM((_NCHUNK, _GCHUNK), jnp.int32),
            pltpu.VMEM((2, _GCHUNK, EMB), jnp.float32),
            pltpu.SemaphoreType.DMA,
            pltpu.SemaphoreType.DMA,
        ],
    )
    def k(weight_hbm, idx_hbm, out_hbm, idx_v, rows_v, sem_in, sem_out):
        wid = lax.axis_index("s") * _NC + lax.axis_index("c")
        base = wid * rows_per_w
        pltpu.sync_copy(idx_hbm.at[wid], idx_v)
        h_in = [None] * _NCHUNK
        h_out = [None] * _NCHUNK
        h_in[0] = pltpu.async_copy(weight_hbm.at[idx_v.at[0]], rows_v.at[0], sem_in)
        for c in range(_NCHUNK):
            p = c % 2
            h_in[c].wait()
            if c + 1 < _NCHUNK:
                if c >= 1:
                    h_out[c - 1].wait()
                h_in[c + 1] = pltpu.async_copy(
                    weight_hbm.at[idx_v.at[c + 1]], rows_v.at[1 - p], sem_in)
            h_out[c] = pltpu.async_copy(
                rows_v.at[p], out_hbm.at[pl.ds(base + c * _GCHUNK, _GCHUNK)], sem_out)
        h_out[_NCHUNK - 2].wait()
        h_out[_NCHUNK - 1].wait()

    return k(weight, idx)


_TBLK = 128


def _tc_tanh(rows):
    n_rows = rows.shape[0]
    grid = (n_rows // _TBLK,)

    def body(in_ref, out_ref, acc_ref):
        p = pl.program_id(0)
        t = jnp.tanh(in_ref[...])
        out_ref[...] = t
        k = lax.broadcasted_iota(jnp.int32, (_TBLK, 1), 0)
        l = (p * _TBLK + k) % SEQ_L
        w = jnp.where(l == SEQ_L - 1, 377.0, 1.0)
        s = jnp.sum(t * t * w)
        acc_ref[0, 0] = jnp.where(p == 0, 0.0, acc_ref[0, 0]) + s

    return pl.pallas_call(
        body,
        grid=grid,
        in_specs=[pl.BlockSpec((_TBLK, EMB), lambda p: (p, 0))],
        out_specs=(
            pl.BlockSpec((_TBLK, EMB), lambda p: (p, 0)),
            pl.BlockSpec((1, 1), lambda p: (0, 0), memory_space=pltpu.SMEM),
        ),
        out_shape=(
            jax.ShapeDtypeStruct((n_rows, EMB), jnp.float32),
            jax.ShapeDtypeStruct((1, 1), jnp.float32),
        ),
    )(rows)


def _sc_scatter(emb2):
    n_per_w = N_BATCH // _NW

    mesh = plsc.VectorSubcoreMesh(core_axis_name="c", subcore_axis_name="s")

    @functools.partial(
        pl.kernel,
        mesh=mesh,
        out_type=jax.ShapeDtypeStruct((N_BATCH, 3, IMG, IMG), jnp.float32),
        scratch_types=[
            pltpu.VMEM((NPR, EMB), jnp.float32),
            pltpu.VMEM((EMB,), jnp.float32),
            pltpu.VMEM((3, PATCH, IMG), jnp.float32),
            pltpu.VMEM((2, 3, PATCH, IMG), jnp.float32),
            pltpu.SemaphoreType.DMA,
            pltpu.SemaphoreType.DMA,
        ],
    )
    def k(emb_hbm, out_hbm, buf, r199, rb199, rb, sem_p, sem_w):
        wid = lax.axis_index("s") * _NC + lax.axis_index("c")
        for dn in range(n_per_w):
            n = wid * n_per_w + dn
            pltpu.sync_copy(emb_hbm.at[n, SEQ_L - 1], r199)

            def build199(j):
                for c in range(3):
                    for pi in range(PATCH):
                        rb199[c, pi, pl.ds(j * PATCH, PATCH)] = (
                            r199[pl.ds(c * 256 + pi * PATCH, PATCH)])

            pl.loop(0, NPR)(build199)

            def fire(i):
                pltpu.async_copy(
                    rb199, out_hbm.at[n, :, pl.ds(i * PATCH, PATCH), :], sem_w)

            pl.loop(9, NPR)(fire)

            prev = [None, None]
            for i in range(9):
                p = i % 2
                nr = NPR if i < 8 else 8
                pltpu.sync_copy(emb_hbm.at[n, pl.ds(i * NPR, nr)],
                                buf.at[pl.ds(0, nr)])
                if prev[p] is not None:
                    prev[p].wait()
                if i == 8:
                    def tail199_p(j, _p=p):
                        for c in range(3):
                            for pi in range(PATCH):
                                rb[_p, c, pi, pl.ds(j * PATCH, PATCH)] = (
                                    r199[pl.ds(c * 256 + pi * PATCH, PATCH)])
                    pl.loop(8, NPR)(tail199_p)

                def interleave(j, _p=p):
                    for c in range(3):
                        for pi in range(PATCH):
                            rb[_p, c, pi, pl.ds(j * PATCH, PATCH)] = (
                                buf[j, pl.ds(c * 256 + pi * PATCH, PATCH)])

                pl.loop(0, nr)(interleave)
                prev[p] = pltpu.async_copy(
                    rb.at[p], out_hbm.at[n, :, pl.ds(i * PATCH, PATCH), :], sem_p)
            prev[0].wait()
            prev[1].wait()

            def drain(i):
                pltpu.make_async_copy(
                    out_hbm.at[n, :, pl.ds(i * PATCH, PATCH), :], rb199, sem_w
                ).wait()

            pl.loop(9, NPR)(drain)

    return k(emb2)


def kernel(sentence_batch, weight):
    n_batch, seq_l = sentence_batch.shape
    tokens = sentence_batch.astype(jnp.int32)
    idx = tokens.reshape(_NW, _NCHUNK, _GCHUNK)
    rows = _sc_gather(weight, idx)
    t_rows, ssq = _tc_tanh(rows)
    img = _sc_scatter(t_rows.reshape(n_batch, seq_l, EMB))
    pert_norm = jnp.sqrt(ssq[0, 0]) / n_batch
    return img, pert_norm
